# baseline (speedup 1.0000x reference)
import dataclasses
import functools

import jax
import jax.numpy as jnp
from jax import lax
from jax.experimental import pallas as pl
from jax.experimental.pallas import tpu as pltpu
from jax.experimental.pallas import tpu_sc as plsc

N = 10000
N_PAD = 10240
IN_CH = 128
HID = 512
OUT_CH = 256
CH2 = 128
E = 320000
E_ROWS = 2560
E_PAD = E_ROWS * 128
N_SUB = 16
ROWS_PER_SUB = N_PAD // N_SUB
ESUB = E_ROWS // N_SUB
ESUB1 = E_ROWS // 32
DSUB = E_ROWS // 32
IDXC = 16
R_BLK = 1280
GRID = N_PAD // R_BLK

_MESH = plsc.VectorSubcoreMesh(core_axis_name="c", subcore_axis_name="s")

_SC_PARAMS = pltpu.CompilerParams()
if "needs_layout_passes" in pltpu.CompilerParams.__dataclass_fields__:
    _SC_PARAMS = dataclasses.replace(_SC_PARAMS, needs_layout_passes=False)


@functools.partial(
    pl.kernel,
    out_type=jax.ShapeDtypeStruct((32, 1, N_PAD), jnp.float32),
    mesh=_MESH,
    compiler_params=_SC_PARAMS,
    scratch_types=[
        pltpu.VMEM((1, N_PAD), jnp.float32),
        pltpu.VMEM((DSUB, 1, 128), jnp.int32),
    ],
)
def _sc_degree(zeros_hbm, dst_hbm, cnt_hbm, cnt_v, idx_v):
    c = lax.axis_index("c")
    s = lax.axis_index("s")
    w = c * N_SUB + s
    pltpu.sync_copy(zeros_hbm, cnt_v)
    pltpu.sync_copy(dst_hbm.at[pl.ds(w * DSUB, DSUB)], idx_v)
    ones = jnp.full((16,), 1.0, jnp.float32)
    zero16 = jnp.zeros((16,), jnp.int32)

    @pl.loop(0, DSUB)
    def _(r):
        @pl.loop(0, 128, step=16)
        def _(k):
            plsc.addupdate_scatter(cnt_v, [zero16, idx_v[r, 0, pl.ds(k, 16)]],
                                   ones)

    pltpu.sync_copy(cnt_v, cnt_hbm.at[w])


@functools.partial(
    pl.kernel,
    out_type=jax.ShapeDtypeStruct((2 * N_PAD, CH2), jnp.float32),
    mesh=_MESH,
    scratch_types=[
        pltpu.VMEM_SHARED((N_PAD, CH2), jnp.float32),
        pltpu.VMEM((IDXC, 1, 128), jnp.int32),
        pltpu.VMEM((IDXC, 1, 128), jnp.int32),
        pltpu.VMEM((128, CH2), jnp.float32),
        pltpu.VMEM((128, CH2), jnp.float32),
        pltpu.SemaphoreType.DMA,
        pltpu.SemaphoreType.DMA,
    ],
)
def _sc_agg2(table_hbm, src_hbm, dst_hbm, out_hbm, acc_sh, sidx, didx,
             buf_a, buf_b, sem_a, sem_b):
    c = lax.axis_index("c")
    s = lax.axis_index("s")
    pltpu.sync_copy(
        table_hbm.at[pl.ds(c * N_PAD + s * ROWS_PER_SUB, ROWS_PER_SUB)],
        acc_sh.at[pl.ds(s * ROWS_PER_SUB, ROWS_PER_SUB)],
    )
    base = s * ESUB
    bufs = (buf_a, buf_b)
    sems = (sem_a, sem_b)
    plsc.subcore_barrier()

    @pl.loop(0, ESUB, step=IDXC)
    def _(jc):
        da = pltpu.async_copy(src_hbm.at[pl.ds(c * E_ROWS + base + jc, IDXC)],
                              sidx, sems[0])
        db = pltpu.async_copy(dst_hbm.at[pl.ds(base + jc, IDXC)], didx,
                              sems[1])
        da.wait()
        db.wait()
        pltpu.async_copy(table_hbm.at[sidx.at[0, 0]], bufs[0], sems[0])
        for j in range(IDXC):
            pltpu.make_async_copy(table_hbm.at[sidx.at[j, 0]],
                                  bufs[j % 2], sems[j % 2]).wait()
            if j + 1 < IDXC:
                pltpu.async_copy(table_hbm.at[sidx.at[j + 1, 0]],
                                 bufs[(j + 1) % 2], sems[(j + 1) % 2])
            pltpu.sync_copy(bufs[j % 2], acc_sh.at[didx.at[j, 0]],
                            add=True)

    plsc.subcore_barrier()
    pltpu.sync_copy(
        acc_sh.at[pl.ds(s * ROWS_PER_SUB, ROWS_PER_SUB)],
        out_hbm.at[pl.ds(c * N_PAD + s * ROWS_PER_SUB, ROWS_PER_SUB)],
    )


@functools.partial(
    pl.kernel,
    out_type=jax.ShapeDtypeStruct((2 * N_PAD, IN_CH), jnp.float32),
    mesh=_MESH,
    scratch_types=[
        pltpu.VMEM_SHARED((N_PAD, IN_CH), jnp.float32),
        pltpu.VMEM((IDXC, 1, 128), jnp.int32),
        pltpu.VMEM((IDXC, 1, 128), jnp.int32),
        pltpu.VMEM((128, IN_CH), jnp.float32),
        pltpu.VMEM((128, IN_CH), jnp.float32),
        pltpu.SemaphoreType.DMA,
        pltpu.SemaphoreType.DMA,
    ],
)
def _sc_agg1(table_hbm, src_hbm, dst_hbm, out_hbm, acc_sh, sidx, didx,
             buf_a, buf_b, sem_a, sem_b):
    c = lax.axis_index("c")
    s = lax.axis_index("s")
    pltpu.sync_copy(
        table_hbm.at[pl.ds(s * ROWS_PER_SUB, ROWS_PER_SUB)],
        acc_sh.at[pl.ds(s * ROWS_PER_SUB, ROWS_PER_SUB)],
    )
    base = ((1 - c) * N_SUB + s) * ESUB1
    bufs = (buf_a, buf_b)
    sems = (sem_a, sem_b)
    plsc.subcore_barrier()

    @pl.loop(0, ESUB1, step=IDXC)
    def _(jc):
        da = pltpu.async_copy(src_hbm.at[pl.ds(base + jc, IDXC)], sidx,
                              sems[0])
        db = pltpu.async_copy(dst_hbm.at[pl.ds(base + jc, IDXC)], didx,
                              sems[1])
        da.wait()
        db.wait()
        pltpu.async_copy(table_hbm.at[sidx.at[0, 0]], bufs[0], sems[0])
        for j in range(IDXC):
            pltpu.make_async_copy(table_hbm.at[sidx.at[j, 0]],
                                  bufs[j % 2], sems[j % 2]).wait()
            if j + 1 < IDXC:
                pltpu.async_copy(table_hbm.at[sidx.at[j + 1, 0]],
                                 bufs[(j + 1) % 2], sems[(j + 1) % 2])
            pltpu.sync_copy(bufs[j % 2], acc_sh.at[didx.at[j, 0]], add=True)

    plsc.subcore_barrier()
    pltpu.sync_copy(
        acc_sh.at[pl.ds(s * ROWS_PER_SUB, ROWS_PER_SUB)],
        out_hbm.at[pl.ds(c * N_PAD + s * ROWS_PER_SUB, ROWS_PER_SUB)],
    )


def _dinv(cnt):
    return lax.rsqrt(jnp.sum(cnt, axis=0) + 1.0)


def _tc_prep(xp, counts):
    def body(x_ref, cnt_ref, out_ref):
        out_ref[...] = x_ref[...] * _dinv(cnt_ref[...])[:, None]

    return pl.pallas_call(
        body,
        grid=(GRID,),
        in_specs=[
            pl.BlockSpec((R_BLK, IN_CH), lambda i: (i, 0)),
            pl.BlockSpec((32, R_BLK), lambda i: (0, i)),
        ],
        out_specs=pl.BlockSpec((R_BLK, IN_CH), lambda i: (i, 0)),
        out_shape=jax.ShapeDtypeStruct((N_PAD, IN_CH), jnp.float32),
    )(xp, counts)


def _tc_mid(agg1, xs, counts, W1, b1, W2):
    def body(a_ref, xs_ref, cnt_ref, w1_ref, b1_ref, w2_ref, out_ref):
        dinv = _dinv(cnt_ref[...])
        a = (a_ref[0] + a_ref[1] - xs_ref[...]) * dinv[:, None]
        h = jnp.dot(a, w1_ref[...], preferred_element_type=jnp.float32)
        h = jnp.maximum(h + b1_ref[...], 0.0)
        hw = jnp.dot(h, w2_ref[...], preferred_element_type=jnp.float32)
        hw = hw * dinv[:, None]
        out_ref[0] = hw[:, :CH2]
        out_ref[1] = hw[:, CH2:]

    return pl.pallas_call(
        body,
        grid=(GRID,),
        in_specs=[
            pl.BlockSpec((2, R_BLK, IN_CH), lambda i: (0, i, 0)),
            pl.BlockSpec((R_BLK, IN_CH), lambda i: (i, 0)),
            pl.BlockSpec((32, R_BLK), lambda i: (0, i)),
            pl.BlockSpec((IN_CH, HID), lambda i: (0, 0)),
            pl.BlockSpec((1, HID), lambda i: (0, 0)),
            pl.BlockSpec((HID, OUT_CH), lambda i: (0, 0)),
        ],
        out_specs=pl.BlockSpec((2, R_BLK, CH2), lambda i: (0, i, 0)),
        out_shape=jax.ShapeDtypeStruct((2, N_PAD, CH2), jnp.float32),
    )(agg1, xs, counts, W1, b1.reshape(1, HID), W2)


def _tc_final(agg2, counts, b2):
    def body(a_ref, cnt_ref, b2_ref, out_ref):
        dinv = _dinv(cnt_ref[...])
        a = jnp.concatenate([a_ref[0], a_ref[1]], axis=1) * dinv[:, None]
        out_ref[...] = jnp.maximum(a + b2_ref[...], 0.0)

    return pl.pallas_call(
        body,
        grid=(GRID,),
        in_specs=[
            pl.BlockSpec((2, R_BLK, CH2), lambda i: (0, i, 0)),
            pl.BlockSpec((32, R_BLK), lambda i: (0, i)),
            pl.BlockSpec((1, OUT_CH), lambda i: (0, 0)),
        ],
        out_specs=pl.BlockSpec((R_BLK, OUT_CH), lambda i: (i, 0)),
        out_shape=jax.ShapeDtypeStruct((N_PAD, OUT_CH), jnp.float32),
    )(agg2, counts, b2.reshape(1, OUT_CH))


def kernel(x, edge_index, W1, b1, W2, b2):
    src = edge_index[0].astype(jnp.int32)
    dst = edge_index[1].astype(jnp.int32)
    padv = jnp.full((E_PAD - E,), N, jnp.int32)
    srcp = jnp.concatenate([src, padv]).reshape(E_ROWS, 1, 128)
    dstp = jnp.concatenate([dst, padv]).reshape(E_ROWS, 1, 128)
    src2 = jnp.concatenate([srcp, srcp + N_PAD], axis=0)
    xp = jnp.pad(x, ((0, N_PAD - N), (0, 0)))

    counts = _sc_degree(jnp.zeros((1, N_PAD), jnp.float32), dstp)
    counts = counts.reshape(32, N_PAD)
    xs = _tc_prep(xp, counts)
    agg1 = _sc_agg1(xs, srcp, dstp).reshape(2, N_PAD, IN_CH)
    hws = _tc_mid(agg1, xs, counts, W1, b1, W2).reshape(2 * N_PAD, CH2)
    agg2 = _sc_agg2(hws, src2, dstp).reshape(2, N_PAD, CH2)
    out = _tc_final(agg2, counts, b2)
    return out[:N]


# speedup vs baseline: 1.0046x; 1.0046x over previous
import dataclasses
import functools

import jax
import jax.numpy as jnp
from jax import lax
from jax.experimental import pallas as pl
from jax.experimental.pallas import tpu as pltpu
from jax.experimental.pallas import tpu_sc as plsc

N = 10000
N_PAD = 10240
IN_CH = 128
HID = 512
OUT_CH = 256
CH2 = 128
E = 320000
E_ROWS = 2560
E_PAD = E_ROWS * 128
N_SUB = 16
ROWS_PER_SUB = N_PAD // N_SUB
ESUB = E_ROWS // N_SUB
ESUB1 = E_ROWS // 32
DSUB = E_ROWS // 32
IDXC = 16
R_BLK = 1280
GRID = N_PAD // R_BLK

_MESH = plsc.VectorSubcoreMesh(core_axis_name="c", subcore_axis_name="s")

_SC_PARAMS = pltpu.CompilerParams()
if "needs_layout_passes" in pltpu.CompilerParams.__dataclass_fields__:
    _SC_PARAMS = dataclasses.replace(_SC_PARAMS, needs_layout_passes=False)


@functools.partial(
    pl.kernel,
    out_type=jax.ShapeDtypeStruct((32, 1, N_PAD), jnp.float32),
    mesh=_MESH,
    compiler_params=_SC_PARAMS,
    scratch_types=[
        pltpu.VMEM((1, N_PAD), jnp.float32),
        pltpu.VMEM((DSUB, 1, 128), jnp.int32),
    ],
)
def _sc_degree(zeros_hbm, dst_hbm, cnt_hbm, cnt_v, idx_v):
    c = lax.axis_index("c")
    s = lax.axis_index("s")
    w = c * N_SUB + s
    pltpu.sync_copy(zeros_hbm, cnt_v)
    pltpu.sync_copy(dst_hbm.at[pl.ds(w * DSUB, DSUB)], idx_v)
    ones = jnp.full((16,), 1.0, jnp.float32)
    zero16 = jnp.zeros((16,), jnp.int32)

    @pl.loop(0, DSUB)
    def _(r):
        @pl.loop(0, 128, step=16)
        def _(k):
            plsc.addupdate_scatter(cnt_v, [zero16, idx_v[r, 0, pl.ds(k, 16)]],
                                   ones)

    pltpu.sync_copy(cnt_v, cnt_hbm.at[w])


@functools.partial(
    pl.kernel,
    out_type=jax.ShapeDtypeStruct((2 * N_PAD, CH2), jnp.float32),
    mesh=_MESH,
    scratch_types=[
        pltpu.VMEM_SHARED((N_PAD, CH2), jnp.float32),
        [pltpu.VMEM((IDXC, 1, 128), jnp.int32)] * 2,
        [pltpu.VMEM((IDXC, 1, 128), jnp.int32)] * 2,
        pltpu.VMEM((128, CH2), jnp.float32),
        pltpu.VMEM((128, CH2), jnp.float32),
        pltpu.SemaphoreType.DMA,
        pltpu.SemaphoreType.DMA,
        pltpu.SemaphoreType.DMA,
        pltpu.SemaphoreType.DMA,
    ],
)
def _sc_agg2(table_hbm, src_hbm, dst_hbm, out_hbm, acc_sh, sidx, didx,
             buf_a, buf_b, sem_a, sem_b, isem_a, isem_b):
    c = lax.axis_index("c")
    s = lax.axis_index("s")
    pltpu.sync_copy(
        table_hbm.at[pl.ds(c * N_PAD + s * ROWS_PER_SUB, ROWS_PER_SUB)],
        acc_sh.at[pl.ds(s * ROWS_PER_SUB, ROWS_PER_SUB)],
    )
    base = s * ESUB
    bufs = (buf_a, buf_b)
    sems = (sem_a, sem_b)
    isems = (isem_a, isem_b)

    def issue_idx(jc, hb):
        pltpu.async_copy(src_hbm.at[pl.ds(c * E_ROWS + base + jc, IDXC)],
                         sidx[hb], isems[0])
        pltpu.async_copy(dst_hbm.at[pl.ds(base + jc, IDXC)], didx[hb],
                         isems[1])

    def wait_idx(hb):
        pltpu.make_async_copy(src_hbm.at[pl.ds(0, IDXC)], sidx[hb],
                              isems[0]).wait()
        pltpu.make_async_copy(dst_hbm.at[pl.ds(0, IDXC)], didx[hb],
                              isems[1]).wait()

    def chunk(sx, dx):
        pltpu.async_copy(table_hbm.at[sx.at[0, 0]], bufs[0], sems[0])
        for j in range(IDXC):
            pltpu.make_async_copy(table_hbm.at[sx.at[j, 0]],
                                  bufs[j % 2], sems[j % 2]).wait()
            if j + 1 < IDXC:
                pltpu.async_copy(table_hbm.at[sx.at[j + 1, 0]],
                                 bufs[(j + 1) % 2], sems[(j + 1) % 2])
            pltpu.sync_copy(bufs[j % 2], acc_sh.at[dx.at[j, 0]], add=True)

    issue_idx(0, 0)
    plsc.subcore_barrier()

    @pl.loop(0, ESUB, step=2 * IDXC)
    def _(jc):
        wait_idx(0)
        issue_idx(jc + IDXC, 1)
        chunk(sidx[0], didx[0])
        wait_idx(1)

        @pl.when(jc + 2 * IDXC < ESUB)
        def _():
            issue_idx(jc + 2 * IDXC, 0)

        chunk(sidx[1], didx[1])

    plsc.subcore_barrier()
    pltpu.sync_copy(
        acc_sh.at[pl.ds(s * ROWS_PER_SUB, ROWS_PER_SUB)],
        out_hbm.at[pl.ds(c * N_PAD + s * ROWS_PER_SUB, ROWS_PER_SUB)],
    )


@functools.partial(
    pl.kernel,
    out_type=jax.ShapeDtypeStruct((2 * N_PAD, IN_CH), jnp.float32),
    mesh=_MESH,
    scratch_types=[
        pltpu.VMEM_SHARED((N_PAD, IN_CH), jnp.float32),
        pltpu.VMEM((IDXC, 1, 128), jnp.int32),
        pltpu.VMEM((IDXC, 1, 128), jnp.int32),
        pltpu.VMEM((128, IN_CH), jnp.float32),
        pltpu.VMEM((128, IN_CH), jnp.float32),
        pltpu.SemaphoreType.DMA,
        pltpu.SemaphoreType.DMA,
    ],
)
def _sc_agg1(table_hbm, src_hbm, dst_hbm, out_hbm, acc_sh, sidx, didx,
             buf_a, buf_b, sem_a, sem_b):
    c = lax.axis_index("c")
    s = lax.axis_index("s")
    pltpu.sync_copy(
        table_hbm.at[pl.ds(s * ROWS_PER_SUB, ROWS_PER_SUB)],
        acc_sh.at[pl.ds(s * ROWS_PER_SUB, ROWS_PER_SUB)],
    )
    base = ((1 - c) * N_SUB + s) * ESUB1
    bufs = (buf_a, buf_b)
    sems = (sem_a, sem_b)
    plsc.subcore_barrier()

    @pl.loop(0, ESUB1, step=IDXC)
    def _(jc):
        da = pltpu.async_copy(src_hbm.at[pl.ds(base + jc, IDXC)], sidx,
                              sems[0])
        db = pltpu.async_copy(dst_hbm.at[pl.ds(base + jc, IDXC)], didx,
                              sems[1])
        da.wait()
        db.wait()
        pltpu.async_copy(table_hbm.at[sidx.at[0, 0]], bufs[0], sems[0])
        for j in range(IDXC):
            pltpu.make_async_copy(table_hbm.at[sidx.at[j, 0]],
                                  bufs[j % 2], sems[j % 2]).wait()
            if j + 1 < IDXC:
                pltpu.async_copy(table_hbm.at[sidx.at[j + 1, 0]],
                                 bufs[(j + 1) % 2], sems[(j + 1) % 2])
            pltpu.sync_copy(bufs[j % 2], acc_sh.at[didx.at[j, 0]], add=True)

    plsc.subcore_barrier()
    pltpu.sync_copy(
        acc_sh.at[pl.ds(s * ROWS_PER_SUB, ROWS_PER_SUB)],
        out_hbm.at[pl.ds(c * N_PAD + s * ROWS_PER_SUB, ROWS_PER_SUB)],
    )


def _dinv(cnt):
    return lax.rsqrt(jnp.sum(cnt, axis=0) + 1.0)


def _tc_prep(xp, counts):
    def body(x_ref, cnt_ref, out_ref):
        out_ref[...] = x_ref[...] * _dinv(cnt_ref[...])[:, None]

    return pl.pallas_call(
        body,
        grid=(GRID,),
        in_specs=[
            pl.BlockSpec((R_BLK, IN_CH), lambda i: (i, 0)),
            pl.BlockSpec((32, R_BLK), lambda i: (0, i)),
        ],
        out_specs=pl.BlockSpec((R_BLK, IN_CH), lambda i: (i, 0)),
        out_shape=jax.ShapeDtypeStruct((N_PAD, IN_CH), jnp.float32),
    )(xp, counts)


def _tc_mid(agg1, xs, counts, W1, b1, W2):
    def body(a_ref, xs_ref, cnt_ref, w1_ref, b1_ref, w2_ref, out_ref):
        dinv = _dinv(cnt_ref[...])
        a = (a_ref[0] + a_ref[1] - xs_ref[...]) * dinv[:, None]
        h = jnp.dot(a, w1_ref[...], preferred_element_type=jnp.float32)
        h = jnp.maximum(h + b1_ref[...], 0.0)
        hw = jnp.dot(h, w2_ref[...], preferred_element_type=jnp.float32)
        hw = hw * dinv[:, None]
        out_ref[0] = hw[:, :CH2]
        out_ref[1] = hw[:, CH2:]

    return pl.pallas_call(
        body,
        grid=(GRID,),
        in_specs=[
            pl.BlockSpec((2, R_BLK, IN_CH), lambda i: (0, i, 0)),
            pl.BlockSpec((R_BLK, IN_CH), lambda i: (i, 0)),
            pl.BlockSpec((32, R_BLK), lambda i: (0, i)),
            pl.BlockSpec((IN_CH, HID), lambda i: (0, 0)),
            pl.BlockSpec((1, HID), lambda i: (0, 0)),
            pl.BlockSpec((HID, OUT_CH), lambda i: (0, 0)),
        ],
        out_specs=pl.BlockSpec((2, R_BLK, CH2), lambda i: (0, i, 0)),
        out_shape=jax.ShapeDtypeStruct((2, N_PAD, CH2), jnp.float32),
    )(agg1, xs, counts, W1, b1.reshape(1, HID), W2)


def _tc_final(agg2, counts, b2):
    def body(a_ref, cnt_ref, b2_ref, out_ref):
        dinv = _dinv(cnt_ref[...])
        a = jnp.concatenate([a_ref[0], a_ref[1]], axis=1) * dinv[:, None]
        out_ref[...] = jnp.maximum(a + b2_ref[...], 0.0)

    return pl.pallas_call(
        body,
        grid=(GRID,),
        in_specs=[
            pl.BlockSpec((2, R_BLK, CH2), lambda i: (0, i, 0)),
            pl.BlockSpec((32, R_BLK), lambda i: (0, i)),
            pl.BlockSpec((1, OUT_CH), lambda i: (0, 0)),
        ],
        out_specs=pl.BlockSpec((R_BLK, OUT_CH), lambda i: (i, 0)),
        out_shape=jax.ShapeDtypeStruct((N_PAD, OUT_CH), jnp.float32),
    )(agg2, counts, b2.reshape(1, OUT_CH))


def kernel(x, edge_index, W1, b1, W2, b2):
    src = edge_index[0].astype(jnp.int32)
    dst = edge_index[1].astype(jnp.int32)
    padv = jnp.full((E_PAD - E,), N, jnp.int32)
    srcp = jnp.concatenate([src, padv]).reshape(E_ROWS, 1, 128)
    dstp = jnp.concatenate([dst, padv]).reshape(E_ROWS, 1, 128)
    src2 = jnp.concatenate([srcp, srcp + N_PAD], axis=0)
    xp = jnp.pad(x, ((0, N_PAD - N), (0, 0)))

    counts = _sc_degree(jnp.zeros((1, N_PAD), jnp.float32), dstp)
    counts = counts.reshape(32, N_PAD)
    xs = _tc_prep(xp, counts)
    agg1 = _sc_agg1(xs, srcp, dstp).reshape(2, N_PAD, IN_CH)
    hws = _tc_mid(agg1, xs, counts, W1, b1, W2).reshape(2 * N_PAD, CH2)
    agg2 = _sc_agg2(hws, src2, dstp).reshape(2, N_PAD, CH2)
    out = _tc_final(agg2, counts, b2)
    return out[:N]
